# Initial kernel scaffold; baseline (speedup 1.0000x reference)
import functools

import jax
import jax.numpy as jnp
from jax import lax
from jax.experimental import pallas as pl
from jax.experimental.pallas import tpu as pltpu
from jax.experimental.pallas import tpu_sc as plsc

_N = 10000
_E = 160000
_D = 256
_H = 256
_NB = 8

_NC = 2
_NS = 16
_NW = _NC * _NS

_EG = _E // _NW
_CG = 40
_GI = _EG // _CG


def _sc_mesh():
    return plsc.VectorSubcoreMesh(core_axis_name="c", subcore_axis_name="s")


@functools.partial(
    pl.kernel,
    out_type=jax.ShapeDtypeStruct((_E, _D), jnp.float32),
    mesh=_sc_mesh(),
    scratch_types=[
        pltpu.VMEM((_CG,), jnp.int32),
        pltpu.VMEM((_CG, _D), jnp.float32),
        pltpu.SemaphoreType.DMA,
    ],
)
def _sc_gather(node_hbm, col_hbm, out_hbm, idx_v, rows_v, sem):
    wid = lax.axis_index("s") * _NC + lax.axis_index("c")
    base = wid * _EG

    def body(i, carry):
        off = base + i * _CG
        pltpu.sync_copy(col_hbm.at[pl.ds(off, _CG)], idx_v)
        pltpu.async_copy(node_hbm.at[idx_v], rows_v, sem).wait()
        pltpu.sync_copy(rows_v, out_hbm.at[pl.ds(off, _CG)])
        return carry

    lax.fori_loop(0, _GI, body, 0)


_NH = _N // _NC
_NHP = 5120
_ZR = _NHP // _NS
_CS = 80
_ET = _E // _NS
_SI = _ET // _CS
_OT = _NH // _NS // 8 * 8


@functools.partial(
    pl.kernel,
    out_type=jax.ShapeDtypeStruct((_N, _D), jnp.float32),
    mesh=_sc_mesh(),
    scratch_types=[
        pltpu.VMEM((_CS,), jnp.int32),
        pltpu.VMEM((_CS,), jnp.int32),
        pltpu.VMEM((_CS, _D), jnp.float32),
        pltpu.VMEM_SHARED((_NHP, _D), jnp.float32),
        pltpu.SemaphoreType.DMA,
    ],
)
def _sc_scatter(msg_hbm, row_hbm, zeros_hbm, out_hbm, raw_v, loc_v, msg_v,
                acc_sh, sem):
    cid = lax.axis_index("c")
    sid = lax.axis_index("s")
    nbase = cid * _NH

    pltpu.sync_copy(zeros_hbm, acc_sh.at[pl.ds(sid * _ZR, _ZR)])
    plsc.subcore_barrier()

    def body(i, carry):
        off = sid * _ET + i * _CS
        pltpu.sync_copy(row_hbm.at[pl.ds(off, _CS)], raw_v)
        pltpu.sync_copy(msg_hbm.at[pl.ds(off, _CS)], msg_v)
        for k in range(_CS // 16):
            r = raw_v[pl.ds(k * 16, 16)]
            local = r - nbase
            ok = (local >= 0) & (local < _NH)
            loc_v[pl.ds(k * 16, 16)] = jnp.where(ok, local, _NH)
        pltpu.sync_copy(msg_v, acc_sh.at[loc_v], add=True)
        return carry

    lax.fori_loop(0, _SI, body, 0)
    plsc.subcore_barrier()

    pltpu.sync_copy(acc_sh.at[pl.ds(sid * _OT, _OT)],
                    out_hbm.at[pl.ds(nbase + sid * _OT, _OT)])

    @pl.when(sid == _NS - 1)
    def _tail():
        rem = _NH - _NS * _OT
        if rem:
            pltpu.sync_copy(acc_sh.at[pl.ds(_NS * _OT, rem)],
                            out_hbm.at[pl.ds(nbase + _NS * _OT, rem)])


_BE = 1600


def _mlp_body(rbf_ref, nj_ref, w1_ref, b1_ref, w2_ref, b2_ref, w3_ref, b3_ref,
              o_ref):
    h = rbf_ref[...] @ w1_ref[...] + b1_ref[...]
    h = h * jax.nn.sigmoid(h)
    h = h @ w2_ref[...] + b2_ref[...]
    h = h * jax.nn.sigmoid(h)
    w = h @ w3_ref[...] + b3_ref[...]
    o_ref[...] = nj_ref[...] * w


def _tc_mlp(rbf, nj, w1, b1, w2, b2, w3f, b3f):
    return pl.pallas_call(
        _mlp_body,
        grid=(_E // _BE,),
        in_specs=[
            pl.BlockSpec((_BE, _NB), lambda i: (i, 0)),
            pl.BlockSpec((_BE, _D), lambda i: (i, 0)),
            pl.BlockSpec((_NB, _H), lambda i: (0, 0)),
            pl.BlockSpec((1, _H), lambda i: (0, 0)),
            pl.BlockSpec((_H, _H), lambda i: (0, 0)),
            pl.BlockSpec((1, _H), lambda i: (0, 0)),
            pl.BlockSpec((_H, _D), lambda i: (0, 0)),
            pl.BlockSpec((1, _D), lambda i: (0, 0)),
        ],
        out_specs=pl.BlockSpec((_BE, _D), lambda i: (i, 0)),
        out_shape=jax.ShapeDtypeStruct((_E, _D), jnp.float32),
    )(rbf, nj, w1, b1.reshape(1, _H), w2, b2.reshape(1, _H), w3f,
      b3f.reshape(1, _D))


_BN = 1000


def _final_body(nf_ref, agg_ref, wa_ref, wb_ref, bl_ref, g_ref, bt_ref, o_ref):
    nf = nf_ref[...]
    upd = nf @ wa_ref[...] + agg_ref[...] @ wb_ref[...] + bl_ref[...]
    x = nf + upd
    mean = jnp.mean(x, axis=-1, keepdims=True)
    xc = x - mean
    var = jnp.mean(xc * xc, axis=-1, keepdims=True)
    o_ref[...] = xc * lax.rsqrt(var + 1e-5) * g_ref[...] + bt_ref[...]


def _tc_final(nf, agg, wa, wb, bl, gamma, beta):
    return pl.pallas_call(
        _final_body,
        grid=(_N // _BN,),
        in_specs=[
            pl.BlockSpec((_BN, _D), lambda i: (i, 0)),
            pl.BlockSpec((_BN, _D), lambda i: (i, 0)),
            pl.BlockSpec((_D, _D), lambda i: (0, 0)),
            pl.BlockSpec((_D, _D), lambda i: (0, 0)),
            pl.BlockSpec((1, _D), lambda i: (0, 0)),
            pl.BlockSpec((1, _D), lambda i: (0, 0)),
            pl.BlockSpec((1, _D), lambda i: (0, 0)),
        ],
        out_specs=pl.BlockSpec((_BN, _D), lambda i: (i, 0)),
        out_shape=jax.ShapeDtypeStruct((_N, _D), jnp.float32),
    )(nf, agg, wa, wb, bl.reshape(1, _D), gamma.reshape(1, _D),
      beta.reshape(1, _D))


def kernel(node_feats, edge_index, edge_rbf, edge_sh, W1, b1, W2, b2, W3, b3,
           Wl, bl, gamma, beta):
    del edge_sh
    col = edge_index[1].astype(jnp.int32)
    row = edge_index[0].astype(jnp.int32)
    w3f = W3.reshape(_H, _D, 3).sum(axis=-1)
    b3f = b3.reshape(_D, 3).sum(axis=-1)
    zeros = jnp.zeros((_ZR, _D), jnp.float32)

    node_j = _sc_gather(node_feats, col)
    messages = _tc_mlp(edge_rbf, node_j, W1, b1, W2, b2, w3f, b3f)
    agg = _sc_scatter(messages, row, zeros)
    return _tc_final(node_feats, agg, Wl[:_D], Wl[_D:], bl, gamma, beta)


# speedup vs baseline: 3.9696x; 3.9696x over previous
import functools

import jax
import jax.numpy as jnp
from jax import lax
from jax.experimental import pallas as pl
from jax.experimental.pallas import tpu as pltpu
from jax.experimental.pallas import tpu_sc as plsc

_N = 10000
_E = 160000
_D = 256
_H = 256
_NB = 8

_NC = 2
_NS = 16
_NW = _NC * _NS

_EG = _E // _NW
_CG = 40
_GI = _EG // _CG


def _sc_mesh():
    return plsc.VectorSubcoreMesh(core_axis_name="c", subcore_axis_name="s")


@functools.partial(
    pl.kernel,
    out_type=jax.ShapeDtypeStruct((_E, _D), jnp.float32),
    mesh=_sc_mesh(),
    scratch_types=[
        pltpu.VMEM((_CG,), jnp.int32),
        pltpu.VMEM((_CG, _D), jnp.float32),
        pltpu.SemaphoreType.DMA,
    ],
)
def _sc_gather(node_hbm, col_hbm, out_hbm, idx_v, rows_v, sem):
    wid = lax.axis_index("s") * _NC + lax.axis_index("c")
    base = wid * _EG

    def body(i, carry):
        off = base + i * _CG
        pltpu.sync_copy(col_hbm.at[pl.ds(off, _CG)], idx_v)
        pltpu.async_copy(node_hbm.at[idx_v], rows_v, sem).wait()
        pltpu.sync_copy(rows_v, out_hbm.at[pl.ds(off, _CG)])
        return carry

    lax.fori_loop(0, _GI, body, 0)


_DH = _D // _NC
_CS = 80
_ET = _E // _NS
_SI = _ET // _CS
_NP = 10240
_ZR = _NP // _NS
_LR = _N - (_NS - 1) * _ZR


@functools.partial(
    pl.kernel,
    out_type=jax.ShapeDtypeStruct((_N, _D), jnp.float32),
    mesh=_sc_mesh(),
    scratch_types=[
        pltpu.VMEM((_CS,), jnp.int32),
        pltpu.VMEM((_CS, _DH), jnp.float32),
        pltpu.VMEM_SHARED((_NP, _DH), jnp.float32),
    ],
)
def _sc_scatter(msg_hbm, row_hbm, zeros_hbm, out_hbm, raw_v, msg_v, acc_sh):
    cid = lax.axis_index("c")
    sid = lax.axis_index("s")
    cbase = pl.multiple_of(cid * _DH, _DH)
    rbase = pl.multiple_of(sid * _ZR, 8)

    pltpu.sync_copy(zeros_hbm, acc_sh.at[pl.ds(rbase, _ZR)])
    plsc.subcore_barrier()

    def body(i, carry):
        off = pl.multiple_of(sid * _ET + i * _CS, 8)
        pltpu.sync_copy(row_hbm.at[pl.ds(off, _CS)], raw_v)
        pltpu.sync_copy(msg_hbm.at[pl.ds(off, _CS), pl.ds(cbase, _DH)],
                        msg_v)
        pltpu.sync_copy(msg_v, acc_sh.at[raw_v], add=True)
        return carry

    lax.fori_loop(0, _SI, body, 0)
    plsc.subcore_barrier()

    @pl.when(sid < _NS - 1)
    def _out_full():
        pltpu.sync_copy(acc_sh.at[pl.ds(rbase, _ZR)],
                        out_hbm.at[pl.ds(rbase, _ZR), pl.ds(cbase, _DH)])

    @pl.when(sid == _NS - 1)
    def _out_tail():
        pltpu.sync_copy(acc_sh.at[pl.ds(rbase, _LR)],
                        out_hbm.at[pl.ds(rbase, _LR), pl.ds(cbase, _DH)])


_BE = 1600


def _mlp_body(rbf_ref, nj_ref, w1_ref, b1_ref, w2_ref, b2_ref, w3_ref, b3_ref,
              o_ref):
    h = rbf_ref[...] @ w1_ref[...] + b1_ref[...]
    h = h * jax.nn.sigmoid(h)
    h = h @ w2_ref[...] + b2_ref[...]
    h = h * jax.nn.sigmoid(h)
    w = h @ w3_ref[...] + b3_ref[...]
    o_ref[...] = nj_ref[...] * w


def _tc_mlp(rbf, nj, w1, b1, w2, b2, w3f, b3f):
    return pl.pallas_call(
        _mlp_body,
        grid=(_E // _BE,),
        in_specs=[
            pl.BlockSpec((_BE, _NB), lambda i: (i, 0)),
            pl.BlockSpec((_BE, _D), lambda i: (i, 0)),
            pl.BlockSpec((_NB, _H), lambda i: (0, 0)),
            pl.BlockSpec((1, _H), lambda i: (0, 0)),
            pl.BlockSpec((_H, _H), lambda i: (0, 0)),
            pl.BlockSpec((1, _H), lambda i: (0, 0)),
            pl.BlockSpec((_H, _D), lambda i: (0, 0)),
            pl.BlockSpec((1, _D), lambda i: (0, 0)),
        ],
        out_specs=pl.BlockSpec((_BE, _D), lambda i: (i, 0)),
        out_shape=jax.ShapeDtypeStruct((_E, _D), jnp.float32),
    )(rbf, nj, w1, b1.reshape(1, _H), w2, b2.reshape(1, _H), w3f,
      b3f.reshape(1, _D))


_BN = 1000


def _final_body(nf_ref, agg_ref, wa_ref, wb_ref, bl_ref, g_ref, bt_ref, o_ref):
    nf = nf_ref[...]
    upd = nf @ wa_ref[...] + agg_ref[...] @ wb_ref[...] + bl_ref[...]
    x = nf + upd
    mean = jnp.mean(x, axis=-1, keepdims=True)
    xc = x - mean
    var = jnp.mean(xc * xc, axis=-1, keepdims=True)
    o_ref[...] = xc * lax.rsqrt(var + 1e-5) * g_ref[...] + bt_ref[...]


def _tc_final(nf, agg, wa, wb, bl, gamma, beta):
    return pl.pallas_call(
        _final_body,
        grid=(_N // _BN,),
        in_specs=[
            pl.BlockSpec((_BN, _D), lambda i: (i, 0)),
            pl.BlockSpec((_BN, _D), lambda i: (i, 0)),
            pl.BlockSpec((_D, _D), lambda i: (0, 0)),
            pl.BlockSpec((_D, _D), lambda i: (0, 0)),
            pl.BlockSpec((1, _D), lambda i: (0, 0)),
            pl.BlockSpec((1, _D), lambda i: (0, 0)),
            pl.BlockSpec((1, _D), lambda i: (0, 0)),
        ],
        out_specs=pl.BlockSpec((_BN, _D), lambda i: (i, 0)),
        out_shape=jax.ShapeDtypeStruct((_N, _D), jnp.float32),
    )(nf, agg, wa, wb, bl.reshape(1, _D), gamma.reshape(1, _D),
      beta.reshape(1, _D))


def kernel(node_feats, edge_index, edge_rbf, edge_sh, W1, b1, W2, b2, W3, b3,
           Wl, bl, gamma, beta):
    del edge_sh
    col = edge_index[1].astype(jnp.int32)
    row = edge_index[0].astype(jnp.int32)
    w3f = W3.reshape(_H, _D, 3).sum(axis=-1)
    b3f = b3.reshape(_D, 3).sum(axis=-1)
    zeros = jnp.zeros((_ZR, _DH), jnp.float32)

    node_j = _sc_gather(node_feats, col)
    messages = _tc_mlp(edge_rbf, node_j, W1, b1, W2, b2, w3f, b3f)
    agg = _sc_scatter(messages, row, zeros)
    return _tc_final(node_feats, agg, Wl[:_D], Wl[_D:], bl, gamma, beta)


# speedup vs baseline: 6.0849x; 1.5329x over previous
import functools

import jax
import jax.numpy as jnp
from jax import lax
from jax.experimental import pallas as pl
from jax.experimental.pallas import tpu as pltpu
from jax.experimental.pallas import tpu_sc as plsc

_N = 10000
_E = 160000
_D = 256
_H = 256
_NB = 8

_NC = 2
_NS = 16
_NW = _NC * _NS

_EG = _E // _NW
_CG = 128
_GF = _EG // _CG
_GT = _EG - _GF * _CG


def _sc_mesh():
    return plsc.VectorSubcoreMesh(core_axis_name="c", subcore_axis_name="s")


@functools.partial(
    pl.kernel,
    out_type=jax.ShapeDtypeStruct((_E, _D), jnp.float32),
    mesh=_sc_mesh(),
    scratch_types=[
        pltpu.VMEM((_CG,), jnp.int32),
        pltpu.VMEM((_CG,), jnp.int32),
        pltpu.VMEM((_CG, _D), jnp.float32),
        pltpu.VMEM((_CG, _D), jnp.float32),
        pltpu.SemaphoreType.DMA,
        pltpu.SemaphoreType.DMA,
    ],
)
def _sc_gather(node_hbm, col_hbm, out_hbm, idx0, idx1, rows0, rows1, sem0,
               sem1):
    wid = lax.axis_index("s") * _NC + lax.axis_index("c")
    base = wid * _EG
    idx = (idx0, idx1)
    rows = (rows0, rows1)
    sem = (sem0, sem1)

    handles = [None, None]
    for i in range(_GF):
        s = i % 2
        off = base + i * _CG
        pltpu.sync_copy(col_hbm.at[pl.ds(off, _CG)], idx[s])
        handles[s] = pltpu.async_copy(node_hbm.at[idx[s]], rows[s], sem[s])
        if i > 0:
            handles[1 - s].wait()
            poff = base + (i - 1) * _CG
            pltpu.sync_copy(rows[1 - s], out_hbm.at[pl.ds(poff, _CG)])
    last = (_GF - 1) % 2
    handles[last].wait()
    pltpu.sync_copy(rows[last],
                    out_hbm.at[pl.ds(base + (_GF - 1) * _CG, _CG)])
    toff = base + _GF * _CG
    pltpu.sync_copy(col_hbm.at[pl.ds(toff, _GT)], idx0.at[pl.ds(0, _GT)])
    pltpu.async_copy(node_hbm.at[idx0.at[pl.ds(0, _GT)]],
                     rows0.at[pl.ds(0, _GT)], sem0).wait()
    pltpu.sync_copy(rows0.at[pl.ds(0, _GT)], out_hbm.at[pl.ds(toff, _GT)])


_DH = _D // _NC
_ET = _E // _NS
_CSF = 128
_SF = _ET // _CSF
_ST = _ET - _SF * _CSF
_NP = 10240
_ZR = _NP // _NS
_LR = _N - (_NS - 1) * _ZR


@functools.partial(
    pl.kernel,
    out_type=jax.ShapeDtypeStruct((_N, _D), jnp.float32),
    mesh=_sc_mesh(),
    scratch_types=[
        pltpu.VMEM((_CSF,), jnp.int32),
        pltpu.VMEM((_CSF,), jnp.int32),
        pltpu.VMEM((_ST,), jnp.int32),
        pltpu.VMEM((_CSF, _DH), jnp.float32),
        pltpu.VMEM((_CSF, _DH), jnp.float32),
        pltpu.VMEM((_ST, _DH), jnp.float32),
        pltpu.SemaphoreType.DMA,
        pltpu.SemaphoreType.DMA,
        pltpu.VMEM_SHARED((_NP, _DH), jnp.float32),
    ],
)
def _sc_scatter(msg_hbm, row_hbm, zeros_hbm, out_hbm, idx0, idx1, idx_t,
                msg0, msg1, msg_t, sem0, sem1, acc_sh):
    cid = lax.axis_index("c")
    sid = lax.axis_index("s")
    cbase = pl.multiple_of(cid * _DH, _DH)
    rbase = pl.multiple_of(sid * _ZR, 8)
    ebase = sid * _ET
    idx = (idx0, idx1)
    msg = (msg0, msg1)
    sem = (sem0, sem1)

    pltpu.sync_copy(zeros_hbm, acc_sh.at[pl.ds(rbase, _ZR)])

    handles = [None, None]
    pltpu.sync_copy(row_hbm.at[pl.ds(ebase, _CSF)], idx0)
    handles[0] = pltpu.async_copy(
        msg_hbm.at[pl.ds(ebase, _CSF), pl.ds(cbase, _DH)], msg0, sem0)
    plsc.subcore_barrier()

    handle_t = None
    for i in range(_SF):
        s = i % 2
        if i + 1 < _SF:
            noff = ebase + (i + 1) * _CSF
            pltpu.sync_copy(row_hbm.at[pl.ds(noff, _CSF)], idx[1 - s])
            handles[1 - s] = pltpu.async_copy(
                msg_hbm.at[pl.ds(noff, _CSF), pl.ds(cbase, _DH)],
                msg[1 - s], sem[1 - s])
        else:
            toff = ebase + _SF * _CSF
            pltpu.sync_copy(row_hbm.at[pl.ds(toff, _ST)], idx_t)
            handle_t = pltpu.async_copy(
                msg_hbm.at[pl.ds(toff, _ST), pl.ds(cbase, _DH)], msg_t,
                sem[1 - s])
        handles[s].wait()
        pltpu.sync_copy(msg[s], acc_sh.at[idx[s]], add=True)
    handle_t.wait()
    pltpu.sync_copy(msg_t, acc_sh.at[idx_t], add=True)
    plsc.subcore_barrier()

    @pl.when(sid < _NS - 1)
    def _out_full():
        pltpu.sync_copy(acc_sh.at[pl.ds(rbase, _ZR)],
                        out_hbm.at[pl.ds(rbase, _ZR), pl.ds(cbase, _DH)])

    @pl.when(sid == _NS - 1)
    def _out_tail():
        pltpu.sync_copy(acc_sh.at[pl.ds(rbase, _LR)],
                        out_hbm.at[pl.ds(rbase, _LR), pl.ds(cbase, _DH)])


_BE = 1600


def _mlp_body(rbf_ref, nj_ref, w1_ref, b1_ref, w2_ref, b2_ref, w3_ref, b3_ref,
              o_ref):
    h = rbf_ref[...] @ w1_ref[...] + b1_ref[...]
    h = h * jax.nn.sigmoid(h)
    h = h @ w2_ref[...] + b2_ref[...]
    h = h * jax.nn.sigmoid(h)
    w = h @ w3_ref[...] + b3_ref[...]
    o_ref[...] = nj_ref[...] * w


def _tc_mlp(rbf, nj, w1, b1, w2, b2, w3f, b3f):
    return pl.pallas_call(
        _mlp_body,
        grid=(_E // _BE,),
        in_specs=[
            pl.BlockSpec((_BE, _NB), lambda i: (i, 0)),
            pl.BlockSpec((_BE, _D), lambda i: (i, 0)),
            pl.BlockSpec((_NB, _H), lambda i: (0, 0)),
            pl.BlockSpec((1, _H), lambda i: (0, 0)),
            pl.BlockSpec((_H, _H), lambda i: (0, 0)),
            pl.BlockSpec((1, _H), lambda i: (0, 0)),
            pl.BlockSpec((_H, _D), lambda i: (0, 0)),
            pl.BlockSpec((1, _D), lambda i: (0, 0)),
        ],
        out_specs=pl.BlockSpec((_BE, _D), lambda i: (i, 0)),
        out_shape=jax.ShapeDtypeStruct((_E, _D), jnp.float32),
    )(rbf, nj, w1, b1.reshape(1, _H), w2, b2.reshape(1, _H), w3f,
      b3f.reshape(1, _D))


_BN = 1000


def _final_body(nf_ref, agg_ref, wa_ref, wb_ref, bl_ref, g_ref, bt_ref, o_ref):
    nf = nf_ref[...]
    upd = nf @ wa_ref[...] + agg_ref[...] @ wb_ref[...] + bl_ref[...]
    x = nf + upd
    mean = jnp.mean(x, axis=-1, keepdims=True)
    xc = x - mean
    var = jnp.mean(xc * xc, axis=-1, keepdims=True)
    o_ref[...] = xc * lax.rsqrt(var + 1e-5) * g_ref[...] + bt_ref[...]


def _tc_final(nf, agg, wa, wb, bl, gamma, beta):
    return pl.pallas_call(
        _final_body,
        grid=(_N // _BN,),
        in_specs=[
            pl.BlockSpec((_BN, _D), lambda i: (i, 0)),
            pl.BlockSpec((_BN, _D), lambda i: (i, 0)),
            pl.BlockSpec((_D, _D), lambda i: (0, 0)),
            pl.BlockSpec((_D, _D), lambda i: (0, 0)),
            pl.BlockSpec((1, _D), lambda i: (0, 0)),
            pl.BlockSpec((1, _D), lambda i: (0, 0)),
            pl.BlockSpec((1, _D), lambda i: (0, 0)),
        ],
        out_specs=pl.BlockSpec((_BN, _D), lambda i: (i, 0)),
        out_shape=jax.ShapeDtypeStruct((_N, _D), jnp.float32),
    )(nf, agg, wa, wb, bl.reshape(1, _D), gamma.reshape(1, _D),
      beta.reshape(1, _D))


def kernel(node_feats, edge_index, edge_rbf, edge_sh, W1, b1, W2, b2, W3, b3,
           Wl, bl, gamma, beta):
    del edge_sh
    col = edge_index[1].astype(jnp.int32)
    row = edge_index[0].astype(jnp.int32)
    w3f = W3.reshape(_H, _D, 3).sum(axis=-1)
    b3f = b3.reshape(_D, 3).sum(axis=-1)
    zeros = jnp.zeros((_ZR, _DH), jnp.float32)

    node_j = _sc_gather(node_feats, col)
    messages = _tc_mlp(edge_rbf, node_j, W1, b1, W2, b2, w3f, b3f)
    agg = _sc_scatter(messages, row, zeros)
    return _tc_final(node_feats, agg, Wl[:_D], Wl[_D:], bl, gamma, beta)
